# baseline (speedup 1.0000x reference)
import functools

import jax
import jax.numpy as jnp
from jax import lax
from jax.experimental import pallas as pl
from jax.experimental.pallas import tpu as pltpu
from jax.experimental.pallas import tpu_sc as plsc

N = 10000
E = 320000
D = 128
NG = 64

NC = 2
NS = 16
EDGES_PER_TILE = E // (NC * NS)
SLAB = 624
TAIL_OFF = SLAB * NS
TAIL = N - TAIL_OFF

_MESH = plsc.VectorSubcoreMesh(core_axis_name="c", subcore_axis_name="s")


def _slab_copy(s, src_ref, dst_ref):
    pltpu.sync_copy(src_ref.at[pl.ds(s * SLAB, SLAB)],
                    dst_ref.at[pl.ds(s * SLAB, SLAB)])

    @pl.when(s == NS - 1)
    def _():
        pltpu.sync_copy(src_ref.at[pl.ds(TAIL_OFF, TAIL)],
                        dst_ref.at[pl.ds(TAIL_OFF, TAIL)])


DCHUNK = 125
DNCHUNK = EDGES_PER_TILE // DCHUNK
DG = 8
DGROUPS = DNCHUNK // DG


def _sc_degree_body(dst4_hbm, zeros16_hbm, out_hbm, idxd, ones_v, acc,
                    isem, ssem_a, ssem_b):
    c = lax.axis_index("c")
    s = lax.axis_index("s")
    w = c * NS + s
    di = pltpu.async_copy(dst4_hbm.at[w], idxd, isem)

    def fill_ones(i, _):
        ones_v[i, :] = jnp.ones((16,), jnp.float32)
        return 0

    lax.fori_loop(0, DCHUNK, fill_ones, 0)
    _slab_copy(s, zeros16_hbm, acc)
    di.wait()
    plsc.subcore_barrier()

    def scat(k, sem):
        return [pltpu.async_copy(ones_v, acc.at[idxd.at[k + b]], sem,
                                 add=True)
                for b in range(DG)]

    def drain(ds):
        for d in ds:
            d.wait()

    ga = scat(0, ssem_a)
    gb = scat(DG, ssem_b)

    def pair(i, _):
        k = 2 * DG * i
        drain(ga)
        scat(k + 2 * DG, ssem_a)
        drain(gb)
        scat(k + 3 * DG, ssem_b)
        return 0

    lax.fori_loop(0, DGROUPS // 2 - 1, pair, 0)
    drain(ga)
    drain(gb)
    plsc.subcore_barrier()
    _slab_copy(s, acc, out_hbm.at[c])


_sc_degree = pl.kernel(
    _sc_degree_body,
    out_type=jax.ShapeDtypeStruct((NC, N, 16), jnp.float32),
    mesh=_MESH,
    scratch_types=[
        pltpu.VMEM((DNCHUNK, DCHUNK), jnp.int32),
        pltpu.VMEM((DCHUNK, 16), jnp.float32),
        pltpu.VMEM_SHARED((N, 16), jnp.float32),
        pltpu.SemaphoreType.DMA,
        pltpu.SemaphoreType.DMA,
        pltpu.SemaphoreType.DMA,
    ],
    compiler_params=pltpu.CompilerParams(use_tc_tiling_on_sc=False),
)


FH = D // NC
SCHUNK = 100
SNCHUNK = E // (NS * SCHUNK)
P = 3
PAIRS = (SNCHUNK - P) // (2 * P)
LEFT = SNCHUNK - (2 * P * PAIRS + P)


def _sc_scatter_body(y_hbm, src3_hbm, dst3_hbm, zeros_hbm, out_hbm,
                     idxs, idxd, rows_a, rows_b, acc,
                     isem, gsem_a, gsem_b, ssem_a, ssem_b):
    c = lax.axis_index("c")
    s = lax.axis_index("s")
    yh = y_hbm.at[c]
    da = pltpu.async_copy(src3_hbm.at[s], idxs, isem)
    db = pltpu.async_copy(dst3_hbm.at[s], idxd, isem)
    _slab_copy(s, zeros_hbm, acc)
    da.wait()
    db.wait()
    plsc.subcore_barrier()

    def gather(k, rows, sem):
        return [pltpu.async_copy(yh.at[idxs.at[k + b]], rows.at[b], sem,
                                 priority=1)
                for b in range(P)]

    def scatter(k, rows, sem):
        return [pltpu.async_copy(rows.at[b], acc.at[idxd.at[k + b]], sem,
                                 add=True)
                for b in range(P)]

    def drain(ds):
        for d in ds:
            d.wait()

    ga = gather(0, rows_a, gsem_a)

    def drain_sb():
        for b in range(P):
            pltpu.make_async_copy(zeros_hbm.at[pl.ds(0, SCHUNK)],
                                  rows_b.at[b], ssem_b).wait()

    def pair(i, _):
        k = 2 * P * i
        drain(ga)
        sa = scatter(k, rows_a, ssem_a)

        @pl.when(i > 0)
        def _():
            drain_sb()

        gb = gather(k + P, rows_b, gsem_b)
        drain(gb)
        sb = scatter(k + P, rows_b, ssem_b)
        drain(sa)
        gather(k + 2 * P, rows_a, gsem_a)
        return 0

    lax.fori_loop(0, PAIRS, pair, 0)
    drain(ga)
    sa = scatter(2 * P * PAIRS, rows_a, ssem_a)
    drain_sb()
    drain(sa)
    for r in range(LEFT):
        k = 2 * P * PAIRS + P + r
        pltpu.async_copy(yh.at[idxs.at[k]], rows_b.at[0], gsem_b).wait()
        pltpu.async_copy(rows_b.at[0], acc.at[idxd.at[k]], ssem_b,
                         add=True).wait()

    plsc.subcore_barrier()
    _slab_copy(s, acc, out_hbm.at[c])


_sc_scatter = pl.kernel(
    _sc_scatter_body,
    out_type=jax.ShapeDtypeStruct((NC, N, FH), jnp.float32),
    mesh=_MESH,
    scratch_types=[
        pltpu.VMEM((SNCHUNK, SCHUNK), jnp.int32),
        pltpu.VMEM((SNCHUNK, SCHUNK), jnp.int32),
        pltpu.VMEM((P, SCHUNK, FH), jnp.float32),
        pltpu.VMEM((P, SCHUNK, FH), jnp.float32),
        pltpu.VMEM_SHARED((N, FH), jnp.float32),
        pltpu.SemaphoreType.DMA,
        pltpu.SemaphoreType.DMA,
        pltpu.SemaphoreType.DMA,
        pltpu.SemaphoreType.DMA,
        pltpu.SemaphoreType.DMA,
    ],
    compiler_params=pltpu.CompilerParams(use_tc_tiling_on_sc=False),
)


BR = 2000
NB = N // BR


def _dinv_block(degp):
    deg = degp[0, :, 0] + degp[1, :, 0] + 1.0
    return lax.rsqrt(deg)


def _store_halves(y_ref, y):
    y_ref[0] = y[:, :FH]
    y_ref[1] = y[:, FH:]


def _msg(p_ref, y_ref):
    return jnp.concatenate(
        [p_ref[0] + y_ref[0], p_ref[1] + y_ref[1]], axis=-1)


def _k1_body(x_ref, w_ref, degp_ref, y_ref):
    dinv = _dinv_block(degp_ref[...])
    xw = jnp.dot(x_ref[...], w_ref[...], preferred_element_type=jnp.float32)
    _store_halves(y_ref, dinv[:, None] * xw)


def _kmid_body(p_ref, y_ref, degp_ref, w_ref, b_ref, out_ref):
    dinv = _dinv_block(degp_ref[...])
    h = dinv[:, None] * _msg(p_ref, y_ref) + b_ref[...]
    h = jnp.maximum(h, 0.0)
    _store_halves(out_ref, dinv[:, None] * jnp.dot(
        h, w_ref[...], preferred_element_type=jnp.float32))


def _k4_body(p_ref, y_ref, degp_ref, b_ref, batch_ref, wl_ref, bl_ref,
             out_ref, pooled):
    i = pl.program_id(0)
    dinv = _dinv_block(degp_ref[...])
    h = dinv[:, None] * _msg(p_ref, y_ref) + b_ref[...]
    seg_ids = lax.broadcasted_iota(jnp.int32, (NG, BR), 0)
    mask = (batch_ref[0, 0, :][None, :] == seg_ids).astype(jnp.float32)
    contrib = jnp.dot(mask, h, preferred_element_type=jnp.float32)

    @pl.when(i == 0)
    def _():
        pooled[...] = jnp.zeros_like(pooled)

    pooled[...] += contrib

    @pl.when(i == NB - 1)
    def _():
        out_ref[...] = jnp.dot(
            pooled[...], wl_ref[...],
            preferred_element_type=jnp.float32) + bl_ref[...]


def _full(shape):
    return pl.BlockSpec(shape, lambda i: (0,) * len(shape))


def _tc_k1(x, W1, degp):
    return pl.pallas_call(
        _k1_body,
        grid=(NB,),
        in_specs=[
            pl.BlockSpec((BR, D), lambda i: (i, 0)),
            _full((D, D)),
            pl.BlockSpec((NC, BR, 16), lambda i: (0, i, 0)),
        ],
        out_specs=pl.BlockSpec((NC, BR, FH), lambda i: (0, i, 0)),
        out_shape=jax.ShapeDtypeStruct((NC, N, FH), jnp.float32),
    )(x, W1, degp)


def _tc_kmid(p, y, degp, W, b):
    return pl.pallas_call(
        _kmid_body,
        grid=(NB,),
        in_specs=[
            pl.BlockSpec((NC, BR, FH), lambda i: (0, i, 0)),
            pl.BlockSpec((NC, BR, FH), lambda i: (0, i, 0)),
            pl.BlockSpec((NC, BR, 16), lambda i: (0, i, 0)),
            _full((D, D)),
            _full((1, D)),
        ],
        out_specs=pl.BlockSpec((NC, BR, FH), lambda i: (0, i, 0)),
        out_shape=jax.ShapeDtypeStruct((NC, N, FH), jnp.float32),
    )(p, y, degp, W, b)


def _tc_k4(p, y, degp, b, batch3, Wl, bl):
    return pl.pallas_call(
        _k4_body,
        grid=(NB,),
        in_specs=[
            pl.BlockSpec((NC, BR, FH), lambda i: (0, i, 0)),
            pl.BlockSpec((NC, BR, FH), lambda i: (0, i, 0)),
            pl.BlockSpec((NC, BR, 16), lambda i: (0, i, 0)),
            _full((1, D)),
            pl.BlockSpec((1, 1, BR), lambda i: (i, 0, 0)),
            _full((D, 1)),
            _full((1, 1)),
        ],
        out_specs=pl.BlockSpec((NG, 1), lambda i: (0, 0)),
        out_shape=jax.ShapeDtypeStruct((NG, 1), jnp.float32),
        scratch_shapes=[pltpu.VMEM((NG, D), jnp.float32)],
    )(p, y, degp, b, batch3, Wl, bl)


@jax.jit
def kernel(x, edge_index, batch, W1, b1, W2, b2, W3, b3, Wl, bl):
    src = edge_index[0]
    dst = edge_index[1]
    zeros16 = jnp.zeros((N, 16), jnp.float32)
    zerosh = jnp.zeros((N, FH), jnp.float32)
    batch3 = batch.reshape(NB, 1, BR)
    b1r = b1.reshape(1, D)
    b2r = b2.reshape(1, D)
    b3r = b3.reshape(1, D)
    blr = bl.reshape(1, 1)

    src3 = src.reshape(NS, SNCHUNK, SCHUNK)
    dst3 = dst.reshape(NS, SNCHUNK, SCHUNK)
    dst4 = dst.reshape(NC * NS, DNCHUNK, DCHUNK)

    degp = _sc_degree(dst4, zeros16)
    y1 = _tc_k1(x, W1, degp)
    p1 = _sc_scatter(y1, src3, dst3, zerosh)
    y2 = _tc_kmid(p1, y1, degp, W2, b1r)
    p2 = _sc_scatter(y2, src3, dst3, zerosh)
    y3 = _tc_kmid(p2, y2, degp, W3, b2r)
    p3 = _sc_scatter(y3, src3, dst3, zerosh)
    return _tc_k4(p3, y3, degp, b3r, batch3, Wl, blr)


# speedup vs baseline: 1.0026x; 1.0026x over previous
import jax
import jax.numpy as jnp
from jax import lax
from jax.experimental import pallas as pl
from jax.experimental.pallas import tpu as pltpu
from jax.experimental.pallas import tpu_sc as plsc

N = 10000
E = 320000
D = 128
NG = 64

NC = 2
NS = 16
EDGES_PER_TILE = E // (NC * NS)
SLAB = 624
TAIL_OFF = SLAB * NS
TAIL = N - TAIL_OFF

_MESH = plsc.VectorSubcoreMesh(core_axis_name="c", subcore_axis_name="s")


def _slab_copy(s, src_ref, dst_ref):
    pltpu.sync_copy(src_ref.at[pl.ds(s * SLAB, SLAB)],
                    dst_ref.at[pl.ds(s * SLAB, SLAB)])

    @pl.when(s == NS - 1)
    def _():
        pltpu.sync_copy(src_ref.at[pl.ds(TAIL_OFF, TAIL)],
                        dst_ref.at[pl.ds(TAIL_OFF, TAIL)])


DCHUNK = 125
DNCHUNK = EDGES_PER_TILE // DCHUNK
DG = 8
DGROUPS = DNCHUNK // DG


def _sc_degree_body(dst4_hbm, zeros16_hbm, out_hbm, idxd, ones_v, acc,
                    isem, ssem_a, ssem_b):
    c = lax.axis_index("c")
    s = lax.axis_index("s")
    w = c * NS + s
    di = pltpu.async_copy(dst4_hbm.at[w], idxd, isem)

    def fill_ones(i, _):
        ones_v[i, :] = jnp.ones((16,), jnp.float32)
        return 0

    lax.fori_loop(0, DCHUNK, fill_ones, 0)
    _slab_copy(s, zeros16_hbm, acc)
    di.wait()
    plsc.subcore_barrier()

    def scat(k, sem):
        return [pltpu.async_copy(ones_v, acc.at[idxd.at[k + b]], sem,
                                 add=True)
                for b in range(DG)]

    def drain(ds):
        for d in ds:
            d.wait()

    ga = scat(0, ssem_a)
    gb = scat(DG, ssem_b)

    def pair(i, _):
        k = 2 * DG * i
        drain(ga)
        scat(k + 2 * DG, ssem_a)
        drain(gb)
        scat(k + 3 * DG, ssem_b)
        return 0

    lax.fori_loop(0, DGROUPS // 2 - 1, pair, 0)
    drain(ga)
    drain(gb)
    plsc.subcore_barrier()
    _slab_copy(s, acc, out_hbm.at[c])


_sc_degree = pl.kernel(
    _sc_degree_body,
    out_type=jax.ShapeDtypeStruct((NC, N, 16), jnp.float32),
    mesh=_MESH,
    scratch_types=[
        pltpu.VMEM((DNCHUNK, DCHUNK), jnp.int32),
        pltpu.VMEM((DCHUNK, 16), jnp.float32),
        pltpu.VMEM_SHARED((N, 16), jnp.float32),
        pltpu.SemaphoreType.DMA,
        pltpu.SemaphoreType.DMA,
        pltpu.SemaphoreType.DMA,
    ],
    compiler_params=pltpu.CompilerParams(use_tc_tiling_on_sc=False),
)


FH = D // NC
SCHUNK = 100
SNCHUNK = E // (NS * SCHUNK)
P = 3
PAIRS = (SNCHUNK - P) // (2 * P)
LEFT = SNCHUNK - (2 * P * PAIRS + P)


def _sc_scatter_body(y_hbm, src3_hbm, dst3_hbm, zeros_hbm, out_hbm,
                     idxs, idxd, rows_a, rows_b, acc,
                     isem, gsem_a, gsem_b, ssem_a, ssem_b):
    c = lax.axis_index("c")
    s = lax.axis_index("s")
    yh = y_hbm.at[c]
    da = pltpu.async_copy(src3_hbm.at[s], idxs, isem)
    db = pltpu.async_copy(dst3_hbm.at[s], idxd, isem)
    _slab_copy(s, zeros_hbm, acc)
    da.wait()
    db.wait()
    plsc.subcore_barrier()

    def gather(k, rows, sem):
        return [pltpu.async_copy(yh.at[idxs.at[k + b]], rows.at[b], sem,
                                 priority=1)
                for b in range(P)]

    def scatter(k, rows, sem):
        return [pltpu.async_copy(rows.at[b], acc.at[idxd.at[k + b]], sem,
                                 add=True)
                for b in range(P)]

    def drain(ds):
        for d in ds:
            d.wait()

    ga = gather(0, rows_a, gsem_a)

    def pair(i, _):
        k = 2 * P * i
        drain(ga)
        sa = scatter(k, rows_a, ssem_a)
        gb = gather(k + P, rows_b, gsem_b)
        drain(gb)
        drain(sa)
        sb = scatter(k + P, rows_b, ssem_b)
        gather(k + 2 * P, rows_a, gsem_a)
        drain(sb)
        return 0

    lax.fori_loop(0, PAIRS, pair, 0)
    drain(ga)
    sa = scatter(2 * P * PAIRS, rows_a, ssem_a)
    drain(sa)
    for r in range(LEFT):
        k = 2 * P * PAIRS + P + r
        pltpu.async_copy(yh.at[idxs.at[k]], rows_b.at[0], gsem_b).wait()
        pltpu.async_copy(rows_b.at[0], acc.at[idxd.at[k]], ssem_b,
                         add=True).wait()

    plsc.subcore_barrier()
    _slab_copy(s, acc, out_hbm.at[c])


_sc_scatter = pl.kernel(
    _sc_scatter_body,
    out_type=jax.ShapeDtypeStruct((NC, N, FH), jnp.float32),
    mesh=_MESH,
    scratch_types=[
        pltpu.VMEM((SNCHUNK, SCHUNK), jnp.int32),
        pltpu.VMEM((SNCHUNK, SCHUNK), jnp.int32),
        pltpu.VMEM((P, SCHUNK, FH), jnp.float32),
        pltpu.VMEM((P, SCHUNK, FH), jnp.float32),
        pltpu.VMEM_SHARED((N, FH), jnp.float32),
        pltpu.SemaphoreType.DMA,
        pltpu.SemaphoreType.DMA,
        pltpu.SemaphoreType.DMA,
        pltpu.SemaphoreType.DMA,
        pltpu.SemaphoreType.DMA,
    ],
    compiler_params=pltpu.CompilerParams(use_tc_tiling_on_sc=False),
)


BR = 2000
NB = N // BR


def _dinv_block(degp):
    deg = degp[0, :, 0] + degp[1, :, 0] + 1.0
    return lax.rsqrt(deg)


def _store_halves(y_ref, y):
    y_ref[0] = y[:, :FH]
    y_ref[1] = y[:, FH:]


def _msg(p_ref, y_ref):
    return jnp.concatenate(
        [p_ref[0] + y_ref[0], p_ref[1] + y_ref[1]], axis=-1)


def _k1_body(x_ref, w_ref, degp_ref, y_ref):
    dinv = _dinv_block(degp_ref[...])
    xw = jnp.dot(x_ref[...], w_ref[...], preferred_element_type=jnp.float32)
    _store_halves(y_ref, dinv[:, None] * xw)


def _kmid_body(p_ref, y_ref, degp_ref, w_ref, b_ref, out_ref):
    dinv = _dinv_block(degp_ref[...])
    h = dinv[:, None] * _msg(p_ref, y_ref) + b_ref[...]
    h = jnp.maximum(h, 0.0)
    _store_halves(out_ref, dinv[:, None] * jnp.dot(
        h, w_ref[...], preferred_element_type=jnp.float32))


def _k4_body(p_ref, y_ref, degp_ref, b_ref, batch_ref, wl_ref, bl_ref,
             out_ref, pooled):
    i = pl.program_id(0)
    dinv = _dinv_block(degp_ref[...])
    h = dinv[:, None] * _msg(p_ref, y_ref) + b_ref[...]
    seg_ids = lax.broadcasted_iota(jnp.int32, (NG, BR), 0)
    mask = (batch_ref[0, 0, :][None, :] == seg_ids).astype(jnp.float32)
    contrib = jnp.dot(mask, h, preferred_element_type=jnp.float32)

    @pl.when(i == 0)
    def _():
        pooled[...] = jnp.zeros_like(pooled)

    pooled[...] += contrib

    @pl.when(i == NB - 1)
    def _():
        out_ref[...] = jnp.dot(
            pooled[...], wl_ref[...],
            preferred_element_type=jnp.float32) + bl_ref[...]


def _full(shape):
    return pl.BlockSpec(shape, lambda i: (0,) * len(shape))


def _tc_k1(x, W1, degp):
    return pl.pallas_call(
        _k1_body,
        grid=(NB,),
        in_specs=[
            pl.BlockSpec((BR, D), lambda i: (i, 0)),
            _full((D, D)),
            pl.BlockSpec((NC, BR, 16), lambda i: (0, i, 0)),
        ],
        out_specs=pl.BlockSpec((NC, BR, FH), lambda i: (0, i, 0)),
        out_shape=jax.ShapeDtypeStruct((NC, N, FH), jnp.float32),
    )(x, W1, degp)


def _tc_kmid(p, y, degp, W, b):
    return pl.pallas_call(
        _kmid_body,
        grid=(NB,),
        in_specs=[
            pl.BlockSpec((NC, BR, FH), lambda i: (0, i, 0)),
            pl.BlockSpec((NC, BR, FH), lambda i: (0, i, 0)),
            pl.BlockSpec((NC, BR, 16), lambda i: (0, i, 0)),
            _full((D, D)),
            _full((1, D)),
        ],
        out_specs=pl.BlockSpec((NC, BR, FH), lambda i: (0, i, 0)),
        out_shape=jax.ShapeDtypeStruct((NC, N, FH), jnp.float32),
    )(p, y, degp, W, b)


def _tc_k4(p, y, degp, b, batch3, Wl, bl):
    return pl.pallas_call(
        _k4_body,
        grid=(NB,),
        in_specs=[
            pl.BlockSpec((NC, BR, FH), lambda i: (0, i, 0)),
            pl.BlockSpec((NC, BR, FH), lambda i: (0, i, 0)),
            pl.BlockSpec((NC, BR, 16), lambda i: (0, i, 0)),
            _full((1, D)),
            pl.BlockSpec((1, 1, BR), lambda i: (i, 0, 0)),
            _full((D, 1)),
            _full((1, 1)),
        ],
        out_specs=pl.BlockSpec((NG, 1), lambda i: (0, 0)),
        out_shape=jax.ShapeDtypeStruct((NG, 1), jnp.float32),
        scratch_shapes=[pltpu.VMEM((NG, D), jnp.float32)],
    )(p, y, degp, b, batch3, Wl, bl)


@jax.jit
def kernel(x, edge_index, batch, W1, b1, W2, b2, W3, b3, Wl, bl):
    src = edge_index[0]
    dst = edge_index[1]
    zeros16 = jnp.zeros((N, 16), jnp.float32)
    zerosh = jnp.zeros((N, FH), jnp.float32)
    batch3 = batch.reshape(NB, 1, BR)
    b1r = b1.reshape(1, D)
    b2r = b2.reshape(1, D)
    b3r = b3.reshape(1, D)
    blr = bl.reshape(1, 1)

    src3 = src.reshape(NS, SNCHUNK, SCHUNK)
    dst3 = dst.reshape(NS, SNCHUNK, SCHUNK)
    dst4 = dst.reshape(NC * NS, DNCHUNK, DCHUNK)

    degp = _sc_degree(dst4, zeros16)
    y1 = _tc_k1(x, W1, degp)
    p1 = _sc_scatter(y1, src3, dst3, zerosh)
    y2 = _tc_kmid(p1, y1, degp, W2, b1r)
    p2 = _sc_scatter(y2, src3, dst3, zerosh)
    y3 = _tc_kmid(p2, y2, degp, W3, b2r)
    p3 = _sc_scatter(y3, src3, dst3, zerosh)
    return _tc_k4(p3, y3, degp, b3r, batch3, Wl, blr)
